# baseline (speedup 1.0000x reference)
import functools

import jax
import jax.numpy as jnp
from jax import lax
from jax.experimental import pallas as pl
from jax.experimental.pallas import tpu as pltpu
from jax.experimental.pallas import tpu_sc as plsc

_B = 16384
_D = 26
_K = 1000
_NU = 2 * _D
_HB = _B // 2
_CH = 1024
_NCH = _HB // _CH
_NW = 32


def _sc_gather(dwt2d, mean_flat, lv_flat, labels_t):
    mesh = plsc.VectorSubcoreMesh(core_axis_name="c", subcore_axis_name="s")

    @functools.partial(
        pl.kernel,
        out_type=[
            jax.ShapeDtypeStruct((_D, 32, _B), jnp.float32),
            jax.ShapeDtypeStruct((_D, _B), jnp.float32),
            jax.ShapeDtypeStruct((_D, _B), jnp.float32),
        ],
        mesh=mesh,
        compiler_params=pltpu.CompilerParams(
            needs_layout_passes=False, use_tc_tiling_on_sc=False),
        scratch_types=(
            [pltpu.VMEM((_D, _K), jnp.float32)]
            + [pltpu.VMEM((_K,), jnp.float32)] * 2
            + [pltpu.VMEM((_HB,), jnp.int32)]
            + [pltpu.VMEM((32,), jnp.int32)]
            + [pltpu.VMEM((_D, 1, _CH), jnp.float32)] * 2
            + [pltpu.VMEM((1, _CH), jnp.float32)] * 2
            + [pltpu.VMEM((1, _CH), jnp.float32)] * 2
            + [pltpu.SemaphoreType.DMA] * 3
        ),
    )
    def k(dwt_hbm, mean_hbm, lv_hbm, labt_hbm,
          dw_hbm, mt_hbm, lt_hbm,
          rows, mrow, lrow, lbuf, ridx,
          ob0, ob1, om0, om1, ol0, ol1,
          sgat, so0, so1):
        obdw = (ob0, ob1)
        obm = (om0, om1)
        obl = (ol0, ol1)
        sout = (so0, so1)

        wid = lax.axis_index("s") * 2 + lax.axis_index("c")
        lanes = lax.iota(jnp.int32, 16)
        splat_i = [jnp.full((16,), i, jnp.int32) for i in range(_D)]

        u_start = (13 * wid) // 8
        u_end = (13 * (wid + 1)) // 8

        def out_slices(j, half, c, s):
            b0 = half * _HB + c * _CH
            return (dw_hbm.at[:, pl.ds(j, 1), pl.ds(b0, _CH)],
                    mt_hbm.at[pl.ds(j, 1), pl.ds(b0, _CH)],
                    lt_hbm.at[pl.ds(j, 1), pl.ds(b0, _CH)])

        def fire_out(j, half, c, s):
            dws, ms, ls = out_slices(j, half, c, s)
            pltpu.async_copy(obdw[s], dws, sout[s])
            pltpu.async_copy(obm[s], ms, sout[s])
            pltpu.async_copy(obl[s], ls, sout[s])

        def wait_out(j, half, c, s):
            dws, ms, ls = out_slices(j, half, c, s)
            pltpu.make_async_copy(obdw[s], dws, sout[s]).wait()
            pltpu.make_async_copy(obm[s], ms, sout[s]).wait()
            pltpu.make_async_copy(obl[s], ls, sout[s]).wait()

        def chunk(j, half, c, s, first_round):
            @pl.when(jnp.logical_not(first_round))
            def _():
                wait_out(j, half, c, s)

            @plsc.parallel_loop(0, _CH // 16, unroll=2)
            def v_body(v):
                idxv = lbuf[pl.ds(c * _CH + v * 16, 16)]
                for i in range(_D):
                    val = plsc.load_gather(rows, [splat_i[i], idxv])
                    obdw[s][i, 0, pl.ds(v * 16, 16)] = val
                obm[s][0, pl.ds(v * 16, 16)] = plsc.load_gather(mrow, [idxv])
                obl[s][0, pl.ds(v * 16, 16)] = plsc.load_gather(lrow, [idxv])
            fire_out(j, half, c, s)

        def unit(u, carry):
            j = u // 2
            half = u - 2 * (u // 2)
            ridx[pl.ds(0, 16)] = lanes * _D + j
            ridx[pl.ds(16, 16)] = (lanes + 16) * _D + j
            pltpu.async_copy(dwt_hbm.at[ridx.at[pl.ds(0, _D)]], rows, sgat)
            pltpu.sync_copy(mean_hbm.at[pl.ds(j * _K, _K)], mrow)
            pltpu.sync_copy(lv_hbm.at[pl.ds(j * _K, _K)], lrow)
            pltpu.sync_copy(
                labt_hbm.at[pl.ds(j * _B + half * _HB, _HB)], lbuf)
            pltpu.make_async_copy(
                dwt_hbm.at[ridx.at[pl.ds(0, _D)]], rows, sgat).wait()

            first = u == u_start
            for cc in range(_NCH // 2):
                chunk(j, half, 2 * cc, 0,
                      jnp.logical_and(first, cc == 0))
                chunk(j, half, 2 * cc + 1, 1,
                      jnp.logical_and(first, cc == 0))
            return carry
        lax.fori_loop(u_start, u_end, unit, 0)

        @pl.when(u_end > u_start)
        def _():
            u_last = u_end - 1
            j = u_last // 2
            half = u_last - 2 * (u_last // 2)
            wait_out(j, half, _NCH - 2, 0)
            wait_out(j, half, _NCH - 1, 1)

    return k(dwt2d, mean_flat, lv_flat, labels_t)


def kernel(labels, mean, log_var, domain_weights):
    labels = labels.astype(jnp.int32)
    labels_t = jnp.transpose(labels).reshape(-1)
    dwp, mt, lt = _sc_gather(
        domain_weights.reshape(_D * _D, _K),
        mean.reshape(-1), log_var.reshape(-1), labels_t)
    means = jnp.transpose(mt)
    log_vars = jnp.transpose(lt)
    dw = jnp.transpose(dwp, (2, 0, 1))[:, :, :_D]
    return (means, log_vars, dw)


# speedup vs baseline: 1.2008x; 1.2008x over previous
import functools

import jax
import jax.numpy as jnp
from jax import lax
from jax.experimental import pallas as pl
from jax.experimental.pallas import tpu as pltpu
from jax.experimental.pallas import tpu_sc as plsc

_B = 16384
_D = 26
_K = 1000
_SPLIT = 8
_NU = _SPLIT * _D
_UB = _B // _SPLIT
_CH = 1024
_NCH = _UB // _CH
_NW = 32


def _sc_gather(dwt2d, mean_flat, lv_flat, labels_t):
    mesh = plsc.VectorSubcoreMesh(core_axis_name="c", subcore_axis_name="s")

    @functools.partial(
        pl.kernel,
        out_type=[
            jax.ShapeDtypeStruct((_D, _D, _B), jnp.float32),
            jax.ShapeDtypeStruct((_D, _B), jnp.float32),
            jax.ShapeDtypeStruct((_D, _B), jnp.float32),
        ],
        mesh=mesh,
        compiler_params=pltpu.CompilerParams(
            needs_layout_passes=False, use_tc_tiling_on_sc=False),
        scratch_types=(
            [pltpu.VMEM((_D, _K), jnp.float32)]
            + [pltpu.VMEM((_K,), jnp.float32)] * 2
            + [pltpu.VMEM((_UB,), jnp.int32)] * 2
            + [pltpu.VMEM((32,), jnp.int32)]
            + [pltpu.VMEM((_D, 1, _CH), jnp.float32)] * 2
            + [pltpu.VMEM((1, _CH), jnp.float32)] * 2
            + [pltpu.VMEM((1, _CH), jnp.float32)] * 2
            + [pltpu.SemaphoreType.DMA] * 5
        ),
    )
    def k(dwt_hbm, mean_hbm, lv_hbm, labt_hbm,
          dw_hbm, mt_hbm, lt_hbm,
          rows, mrow, lrow, lb0, lb1, ridx,
          ob0, ob1, om0, om1, ol0, ol1,
          sgat, sla0, sla1, so0, so1):
        lbuf = (lb0, lb1)
        slab = (sla0, sla1)
        obdw = (ob0, ob1)
        obm = (om0, om1)
        obl = (ol0, ol1)
        sout = (so0, so1)

        wid = lax.axis_index("s") * 2 + lax.axis_index("c")
        lanes = lax.iota(jnp.int32, 16)
        splat_i = [jnp.full((16,), i, jnp.int32) for i in range(_D)]

        u_start = (_NU * wid) // _NW
        u_end = (_NU * (wid + 1)) // _NW

        def lab_slice(u):
            j = u // _SPLIT
            sp = u - _SPLIT * j
            return labt_hbm.at[pl.ds(j * _B + sp * _UB, _UB)]

        def fire_labels(u, p):
            pltpu.async_copy(lab_slice(u), lbuf[p], slab[p])

        def wait_labels(u, p):
            pltpu.make_async_copy(lab_slice(u), lbuf[p], slab[p]).wait()

        def stage_rows(j):
            ridx[pl.ds(0, 16)] = lanes * _D + j
            ridx[pl.ds(16, 16)] = (lanes + 16) * _D + j
            pltpu.async_copy(dwt_hbm.at[ridx.at[pl.ds(0, _D)]], rows, sgat)
            pltpu.sync_copy(mean_hbm.at[pl.ds(j * _K, _K)], mrow)
            pltpu.sync_copy(lv_hbm.at[pl.ds(j * _K, _K)], lrow)
            pltpu.make_async_copy(
                dwt_hbm.at[ridx.at[pl.ds(0, _D)]], rows, sgat).wait()

        def out_slices(j, sp, c, s):
            b0 = sp * _UB + c * _CH
            return (dw_hbm.at[:, pl.ds(j, 1), pl.ds(b0, _CH)],
                    mt_hbm.at[pl.ds(j, 1), pl.ds(b0, _CH)],
                    lt_hbm.at[pl.ds(j, 1), pl.ds(b0, _CH)])

        def fire_out(j, sp, c, s):
            dws, ms, ls = out_slices(j, sp, c, s)
            pltpu.async_copy(obdw[s], dws, sout[s])
            pltpu.async_copy(obm[s], ms, sout[s])
            pltpu.async_copy(obl[s], ls, sout[s])

        def wait_out(j, sp, c, s):
            dws, ms, ls = out_slices(j, sp, c, s)
            pltpu.make_async_copy(obdw[s], dws, sout[s]).wait()
            pltpu.make_async_copy(obm[s], ms, sout[s]).wait()
            pltpu.make_async_copy(obl[s], ls, sout[s]).wait()

        def chunk(j, sp, c, s, lp, first_round):
            @pl.when(jnp.logical_not(first_round))
            def _():
                wait_out(j, sp, c, s)

            @plsc.parallel_loop(0, _CH // 16, unroll=2)
            def v_body(v):
                idxv = lbuf[lp][pl.ds(c * _CH + v * 16, 16)]
                for i in range(_D):
                    val = plsc.load_gather(rows, [splat_i[i], idxv])
                    obdw[s][i, 0, pl.ds(v * 16, 16)] = val
                obm[s][0, pl.ds(v * 16, 16)] = plsc.load_gather(mrow, [idxv])
                obl[s][0, pl.ds(v * 16, 16)] = plsc.load_gather(lrow, [idxv])
            fire_out(j, sp, c, s)

        fire_labels(u_start, 0)

        def halfunit(u, lp, prev_j, first):
            j = u // _SPLIT
            sp = u - _SPLIT * j

            @pl.when(j != prev_j)
            def _():
                stage_rows(j)
            wait_labels(u, lp)

            @pl.when(u + 1 < u_end)
            def _():
                fire_labels(u + 1, 1 - lp)
            chunk(j, sp, 0, 0, lp, first)
            chunk(j, sp, 1, 1, lp, first)

        def pairbody(gg, prev_j):
            u = u_start + 2 * gg
            halfunit(u, 0, prev_j, gg == 0)
            j0 = u // _SPLIT

            @pl.when(u + 1 < u_end)
            def _():
                halfunit(u + 1, 1, j0, False)
            j1 = (u + 1) // _SPLIT
            return jnp.where(u + 1 < u_end, j1, j0)
        lax.fori_loop(0, (u_end - u_start + 1) // 2, pairbody,
                      jnp.int32(-1))

        u_last = u_end - 1
        j = u_last // _SPLIT
        sp = u_last - _SPLIT * j
        wait_out(j, sp, 0, 0)
        wait_out(j, sp, 1, 1)

    return k(dwt2d, mean_flat, lv_flat, labels_t)


def kernel(labels, mean, log_var, domain_weights):
    labels = labels.astype(jnp.int32)
    labels_t = jnp.transpose(labels).reshape(-1)
    dwp, mt, lt = _sc_gather(
        domain_weights.reshape(_D * _D, _K),
        mean.reshape(-1), log_var.reshape(-1), labels_t)
    means = jnp.transpose(mt)
    log_vars = jnp.transpose(lt)
    dw = jnp.transpose(dwp, (2, 0, 1))
    return (means, log_vars, dw)
